# Initial kernel scaffold; baseline (speedup 1.0000x reference)
import jax
import jax.numpy as jnp
from jax.experimental import pallas as pl


def kernel(pts, cat_id, params):
    raise NotImplementedError("write your pallas kernel here")


# speedup vs baseline: 6.9396x; 6.9396x over previous
import functools

import jax
import jax.numpy as jnp
from jax import lax
from jax.experimental import pallas as pl

F32 = jnp.float32
I32 = jnp.int32
BS = 8
V0 = 1024
NEG_INF = float("-inf")
_HI_MASK = -65536


def _mm(a, b):
    return lax.dot_general(a, b, (((1,), (0,)), ((), ())),
                           preferred_element_type=F32)


def _split3(x):
    xi = lax.bitcast_convert_type(x, I32)
    p1 = lax.bitcast_convert_type(xi & _HI_MASK, F32)
    r1 = x - p1
    r1i = lax.bitcast_convert_type(r1, I32)
    p2 = lax.bitcast_convert_type(r1i & _HI_MASK, F32)
    r2 = r1 - p2
    return p1, p2, r2


def _gather_exact(oh, parts):
    p1, p2, r2 = parts
    return (_mm(oh, p1) + _mm(oh, p2)) + _mm(oh, r2)


def _topk_idx(neg, k, lanes):
    big = jnp.int32(2 ** 30)
    cols = []
    work = neg
    for _ in range(k):
        m = jnp.max(work, axis=1, keepdims=True)
        arg = jnp.min(jnp.where(work == m, lanes, big), axis=1, keepdims=True)
        cols.append(arg)
        work = jnp.where(lanes == arg, NEG_INF, work)
    return jnp.concatenate(cols, axis=1)


def _argmin_row(d, lanes):
    big = jnp.int32(2 ** 30)
    m = jnp.min(d, axis=1, keepdims=True)
    return jnp.min(jnp.where(d == m, lanes, big), axis=1, keepdims=True)


def _k1_body(verts_ref, vertsT_ref, qc_ref, qr_ref, ni_ref, npi_ref):
    verts = verts_ref[0]
    vertsT = vertsT_ref[0]
    qc = qc_ref[0]
    qr = qr_ref[0]
    inner = _mm(verts, vertsT)
    lanes = lax.broadcasted_iota(I32, (V0, V0), 1)
    dist = -2.0 * inner + qr
    dist = dist + qc
    ni_ref[0] = _topk_idx(-dist, 11, lanes)
    d1 = (qr[:, :256] + qc) - 2.0 * inner[:, :256]
    a1 = _argmin_row(d1, lanes[:, :256])
    d2 = (qr[:, :64] + qc) - 2.0 * inner[:, :64]
    a2 = _argmin_row(d2, lanes[:, :64])
    npi_ref[0] = jnp.concatenate([a1, a2], axis=1)


def _knn_big(verts, vertsT, qc, qr):
    outs = [
        jax.ShapeDtypeStruct((BS, V0, 11), I32),
        jax.ShapeDtypeStruct((BS, V0, 2), I32),
    ]
    return pl.pallas_call(
        _k1_body,
        grid=(BS,),
        in_specs=[
            pl.BlockSpec((1, V0, 3), lambda b: (b, 0, 0)),
            pl.BlockSpec((1, 3, V0), lambda b: (b, 0, 0)),
            pl.BlockSpec((1, V0, 1), lambda b: (b, 0, 0)),
            pl.BlockSpec((1, 1, V0), lambda b: (b, 0, 0)),
        ],
        out_specs=[
            pl.BlockSpec((1, V0, 11), lambda b: (b, 0, 0)),
            pl.BlockSpec((1, V0, 2), lambda b: (b, 0, 0)),
        ],
        out_shape=outs,
    )(verts, vertsT, qc, qr)


def _knn_small_body(v_ref, vT_ref, qc_ref, qr_ref, ni_ref, *, V, K):
    verts = v_ref[0]
    vertsT = vT_ref[0]
    inner = _mm(verts, vertsT)
    lanes = lax.broadcasted_iota(I32, (V, V), 1)
    dist = -2.0 * inner + qr_ref[0]
    dist = dist + qc_ref[0]
    ni_ref[0] = _topk_idx(-dist, K, lanes)


def _knn_small(v, vT, qc, qr, K):
    V = v.shape[1]
    return pl.pallas_call(
        functools.partial(_knn_small_body, V=V, K=K),
        grid=(BS,),
        in_specs=[
            pl.BlockSpec((1, V, 3), lambda b: (b, 0, 0)),
            pl.BlockSpec((1, 3, V), lambda b: (b, 0, 0)),
            pl.BlockSpec((1, V, 1), lambda b: (b, 0, 0)),
            pl.BlockSpec((1, 1, V), lambda b: (b, 0, 0)),
        ],
        out_specs=pl.BlockSpec((1, V, K), lambda b: (b, 0, 0)),
        out_shape=jax.ShapeDtypeStruct((BS, V, K), I32),
    )(v, vT, qc, qr)


def _k2_body(verts_ref, ni_ref, dir0_ref, dir1_ref, w1_ref, b1_ref,
             fm0_ref, fm1_ref):
    V, N = V0, 10
    verts = verts_ref[0]
    ni = ni_ref[0]
    d0 = dir0_ref[...]
    sdn0 = d0 / jnp.maximum(
        jnp.sqrt(jnp.sum(d0 * d0, axis=0, keepdims=True)), 1e-12)
    d1 = dir1_ref[...]
    sdn1 = d1 / jnp.maximum(
        jnp.sqrt(jnp.sum(d1 * d1, axis=0, keepdims=True)), 1e-12)
    lanes = lax.broadcasted_iota(I32, (V, V), 1)
    vparts = _split3(verts)
    ndns = []
    acc0 = jnp.full((V, 896), NEG_INF, F32)
    for j in range(N):
        idx = ni[:, j + 1:j + 2]
        oh = jnp.where(lanes == idx, 1.0, 0.0).astype(F32)
        nb = _gather_exact(oh, vparts)
        diff = nb - verts
        nrm = jnp.sqrt(jnp.sum(diff * diff, axis=1, keepdims=True))
        ndn = diff / jnp.maximum(nrm, 1e-12)
        ndns.append(ndn)
        acc0 = jnp.maximum(acc0, jnp.maximum(_mm(ndn, sdn0), 0.0))
    ssum = acc0[:, 0:128]
    for s in range(1, 7):
        ssum = ssum + acc0[:, s * 128:(s + 1) * 128]
    fm0 = jnp.maximum(ssum, 0.0)
    fm0_ref[0] = fm0
    w1 = w1_ref[...]
    b1 = b1_ref[...]
    center = _mm(fm0, w1[:, 0:128]) + b1[:, 0:128]
    acc1 = jnp.full((V, 896), NEG_INF, F32)
    for j in range(N):
        idx = ni[:, j + 1:j + 2]
        oh = jnp.where(lanes == idx, 1.0, 0.0).astype(F32)
        gf = _mm(oh, fm0)
        sup = _mm(gf, w1[:, 128:]) + b1[:, 128:]
        th = jnp.maximum(_mm(ndns[j], sdn1), 0.0)
        acc1 = jnp.maximum(acc1, th * sup)
    ssum1 = acc1[:, 0:128]
    for s in range(1, 7):
        ssum1 = ssum1 + acc1[:, s * 128:(s + 1) * 128]
    fm1_ref[0] = center + ssum1


def _k2(verts, ni, dir0, dir1, w1, b1):
    outs = [
        jax.ShapeDtypeStruct((BS, V0, 128), F32),
        jax.ShapeDtypeStruct((BS, V0, 128), F32),
    ]
    return pl.pallas_call(
        _k2_body,
        grid=(BS,),
        in_specs=[
            pl.BlockSpec((1, V0, 3), lambda b: (b, 0, 0)),
            pl.BlockSpec((1, V0, 11), lambda b: (b, 0, 0)),
            pl.BlockSpec(dir0.shape, lambda b: (0, 0)),
            pl.BlockSpec(dir1.shape, lambda b: (0, 0)),
            pl.BlockSpec(w1.shape, lambda b: (0, 0)),
            pl.BlockSpec(b1.shape, lambda b: (0, 0)),
        ],
        out_specs=[
            pl.BlockSpec((1, V0, 128), lambda b: (b, 0, 0)),
            pl.BlockSpec((1, V0, 128), lambda b: (b, 0, 0)),
        ],
        out_shape=outs,
    )(verts, ni, dir0, dir1, w1, b1)


def _conv_body(ni_ref, v_ref, f_ref, w_ref, b_ref, dir_ref, *out_refs,
               V, N, Cout, emit_global):
    verts = v_ref[0]
    ni = ni_ref[0]
    f = f_ref[0]
    w = w_ref[...]
    b = b_ref[...]
    dr = dir_ref[...]
    sdn = dr / jnp.maximum(
        jnp.sqrt(jnp.sum(dr * dr, axis=0, keepdims=True)), 1e-12)
    lanes = lax.broadcasted_iota(I32, (V, V), 1)
    vparts = _split3(verts)
    center = _mm(f, w[:, :Cout]) + b[:, :Cout]
    acc = jnp.full((V, 7 * Cout), NEG_INF, F32)
    for j in range(N):
        idx = ni[:, j + 1:j + 2]
        oh = jnp.where(lanes == idx, 1.0, 0.0).astype(F32)
        nb = _gather_exact(oh, vparts)
        diff = nb - verts
        nrm = jnp.sqrt(jnp.sum(diff * diff, axis=1, keepdims=True))
        ndn = diff / jnp.maximum(nrm, 1e-12)
        th = jnp.maximum(_mm(ndn, sdn), 0.0)
        gf = _mm(oh, f)
        sup = _mm(gf, w[:, Cout:]) + b[:, Cout:]
        acc = jnp.maximum(acc, th * sup)
    ssum = acc[:, 0:Cout]
    for s in range(1, 7):
        ssum = ssum + acc[:, s * Cout:(s + 1) * Cout]
    out = center + ssum
    out_refs[0][0] = out
    if emit_global:
        out_refs[1][0] = jnp.max(out, axis=0, keepdims=True)


def _conv(ni, v, f, w, b, dirs, N, Cout, emit_global=False):
    V = v.shape[1]
    K = ni.shape[2]
    Cin = f.shape[2]
    outs = [jax.ShapeDtypeStruct((BS, V, Cout), F32)]
    out_specs = [pl.BlockSpec((1, V, Cout), lambda bb: (bb, 0, 0))]
    if emit_global:
        outs.append(jax.ShapeDtypeStruct((BS, 1, Cout), F32))
        out_specs.append(pl.BlockSpec((1, 1, Cout), lambda bb: (bb, 0, 0)))
    res = pl.pallas_call(
        functools.partial(_conv_body, V=V, N=N, Cout=Cout,
                          emit_global=emit_global),
        grid=(BS,),
        in_specs=[
            pl.BlockSpec((1, V, K), lambda bb: (bb, 0, 0)),
            pl.BlockSpec((1, V, 3), lambda bb: (bb, 0, 0)),
            pl.BlockSpec((1, V, Cin), lambda bb: (bb, 0, 0)),
            pl.BlockSpec(w.shape, lambda bb: (0, 0)),
            pl.BlockSpec(b.shape, lambda bb: (0, 0)),
            pl.BlockSpec(dirs.shape, lambda bb: (0, 0)),
        ],
        out_specs=out_specs,
        out_shape=outs,
    )(ni, v, f, w, b, dirs)
    return res if emit_global else res[0]


def _pool_body(fm_ref, ni_ref, o_ref, *, V, P, C):
    fm = fm_ref[0]
    ni = ni_ref[0]
    lanes = lax.broadcasted_iota(I32, (P, V), 1)
    parts = _split3(fm)
    acc = jnp.full((P, C), NEG_INF, F32)
    for j in range(4):
        idx = ni[0:P, j + 1:j + 2]
        oh = jnp.where(lanes == idx, 1.0, 0.0).astype(F32)
        acc = jnp.maximum(acc, _gather_exact(oh, parts))
    o_ref[0] = acc


def _pool(fm, ni):
    V, C = fm.shape[1], fm.shape[2]
    P = V // 4
    K = ni.shape[2]
    return pl.pallas_call(
        functools.partial(_pool_body, V=V, P=P, C=C),
        grid=(BS,),
        in_specs=[
            pl.BlockSpec((1, V, C), lambda b: (b, 0, 0)),
            pl.BlockSpec((1, V, K), lambda b: (b, 0, 0)),
        ],
        out_specs=pl.BlockSpec((1, P, C), lambda b: (b, 0, 0)),
        out_shape=jax.ShapeDtypeStruct((BS, P, C), F32),
    )(fm, ni)


def _bn_relu_body(x_ref, m_ref, v_ref, g_ref, b_ref, o_ref):
    y = g_ref[...] * (x_ref[...] - m_ref[...]) / jnp.sqrt(v_ref[...] + 1e-5) \
        + b_ref[...]
    o_ref[...] = jnp.maximum(y, 0.0)


def _bn_relu(x, m, var, g, b):
    return pl.pallas_call(
        _bn_relu_body,
        out_shape=jax.ShapeDtypeStruct(x.shape, F32),
    )(x, m, var, g.reshape(1, -1), b.reshape(1, -1))


def _bn_last_stats(x3):
    m = jnp.mean(x3, axis=(0, 1), keepdims=True)
    var = jnp.var(x3, axis=(0, 1), keepdims=True)
    C = x3.shape[2]
    return m.reshape(1, C), var.reshape(1, C)


def _bn_mid_stats(x3):
    xt = jnp.transpose(x3, (0, 2, 1))
    m = jnp.mean(xt, axis=(0, 2), keepdims=True)
    var = jnp.var(xt, axis=(0, 2), keepdims=True)
    C = x3.shape[2]
    return m.reshape(1, C), var.reshape(1, C)


def _head_a_body(fm0_ref, fm1_ref, fm23_ref, fm4_ref, npi_ref,
                 oh6_ref, w_ref, cb_ref, o_ref):
    W = w_ref[...]
    fm0 = fm0_ref[0]
    fm1 = fm1_ref[0]
    fm23 = fm23_ref[0]
    fm4 = fm4_ref[0]
    npi = npi_ref[0]
    lanes256 = lax.broadcasted_iota(I32, (V0, 256), 1)
    lanes64 = lax.broadcasted_iota(I32, (V0, 64), 1)
    oh1 = jnp.where(lanes256 == npi[:, 0:1], 1.0, 0.0).astype(F32)
    oh2 = jnp.where(lanes64 == npi[:, 1:2], 1.0, 0.0).astype(F32)
    g23 = _gather_exact(oh1, _split3(fm23))
    g4 = _gather_exact(oh2, _split3(fm4))
    ohb = jnp.broadcast_to(oh6_ref[0], (V0, 6))
    feat = jnp.concatenate([fm0, fm1, g23, g4, ohb], axis=1)
    o_ref[0] = _mm(feat, W) + cb_ref[...]


def _head_a(fm0, fm1, fm23, fm4, npi, oh6, wT, cb):
    return pl.pallas_call(
        _head_a_body,
        grid=(BS,),
        in_specs=[
            pl.BlockSpec((1, V0, 128), lambda b: (b, 0, 0)),
            pl.BlockSpec((1, V0, 128), lambda b: (b, 0, 0)),
            pl.BlockSpec((1, 256, 512), lambda b: (b, 0, 0)),
            pl.BlockSpec((1, 64, 512), lambda b: (b, 0, 0)),
            pl.BlockSpec((1, V0, 2), lambda b: (b, 0, 0)),
            pl.BlockSpec((1, 1, 6), lambda b: (b, 0, 0)),
            pl.BlockSpec(wT.shape, lambda b: (0, 0)),
            pl.BlockSpec(cb.shape, lambda b: (0, 0)),
        ],
        out_specs=pl.BlockSpec((1, V0, 512), lambda b: (b, 0, 0)),
        out_shape=jax.ShapeDtypeStruct((BS, V0, 512), F32),
    )(fm0, fm1, fm23, fm4, npi, oh6, wT, cb)


def _mmb_body(x_ref, w_ref, b_ref, o_ref):
    o_ref[0] = _mm(x_ref[0], w_ref[...]) + b_ref[...]


def _mmb(x, wT, b):
    BSV, V, Cin = x.shape
    Cout = wT.shape[1]
    return pl.pallas_call(
        _mmb_body,
        grid=(BSV,),
        in_specs=[
            pl.BlockSpec((1, V, Cin), lambda bb: (bb, 0, 0)),
            pl.BlockSpec(wT.shape, lambda bb: (0, 0)),
            pl.BlockSpec(b.shape, lambda bb: (0, 0)),
        ],
        out_specs=pl.BlockSpec((1, V, Cout), lambda bb: (bb, 0, 0)),
        out_shape=jax.ShapeDtypeStruct((BSV, V, Cout), F32),
    )(x, wT, b)


def _head_c_body(fg_ref, x3_ref, v_ref, w_ref, kb_ref, o_ref):
    fgb = jnp.broadcast_to(fg_ref[0], (V0, 512))
    y = jnp.concatenate([fgb, x3_ref[0], v_ref[0]], axis=1)
    o_ref[0] = _mm(y, w_ref[...]) + kb_ref[...]


def _head_c(fg, x3, v, wT, kb):
    return pl.pallas_call(
        _head_c_body,
        grid=(BS,),
        in_specs=[
            pl.BlockSpec((1, 1, 512), lambda b: (b, 0, 0)),
            pl.BlockSpec((1, V0, 256), lambda b: (b, 0, 0)),
            pl.BlockSpec((1, V0, 3), lambda b: (b, 0, 0)),
            pl.BlockSpec(wT.shape, lambda b: (0, 0)),
            pl.BlockSpec(kb.shape, lambda b: (0, 0)),
        ],
        out_specs=pl.BlockSpec((1, V0, 512), lambda b: (b, 0, 0)),
        out_shape=jax.ShapeDtypeStruct((BS, V0, 512), F32),
    )(fg, x3, v, wT, kb)


def _head_d_body(y3_ref, w_ref, b_ref, o_ref):
    o_ref[0] = _mm(y3_ref[0][0:1, :], w_ref[...]) + b_ref[...]


def _head_d(y3, wT, b):
    return pl.pallas_call(
        _head_d_body,
        grid=(BS,),
        in_specs=[
            pl.BlockSpec((1, V0, 128), lambda bb: (bb, 0, 0)),
            pl.BlockSpec(wT.shape, lambda bb: (0, 0)),
            pl.BlockSpec(b.shape, lambda bb: (0, 0)),
        ],
        out_specs=pl.BlockSpec((1, 1, 40), lambda bb: (bb, 0, 0)),
        out_shape=jax.ShapeDtypeStruct((BS, 1, 40), F32),
    )(y3, wT, b)


def kernel(pts, cat_id, params):
    p = params
    mean = jnp.mean(pts, axis=1, keepdims=True)
    verts = pts - mean
    vertsT = jnp.transpose(verts, (0, 2, 1))
    quad = jnp.sum(verts ** 2, axis=2)
    qc = quad[:, :, None]
    qr = quad[:, None, :]
    ni11, npi = _knn_big(verts, vertsT, qc, qr)

    fm0, fm1p = _k2(verts, ni11, p['dir0'], p['dir1'], p['w1'],
                    p['b1'].reshape(1, -1))
    m, var = _bn_last_stats(fm1p)
    fm1 = _bn_relu(fm1p.reshape(BS * V0, 128), m, var, p['bn1_g'], p['bn1_b'])
    fm1 = fm1.reshape(BS, V0, 128)

    f1 = _pool(fm1, ni11)
    v1 = verts[:, :256, :]
    v1T = vertsT[:, :, :256]
    ni2 = _knn_small(v1, v1T, qc[:, :256], qr[:, :, :256], K=11)

    fm2p = _conv(ni2, v1, f1, p['w2'], p['b2'].reshape(1, -1), p['dir2'],
                 N=10, Cout=256)
    m, var = _bn_last_stats(fm2p)
    fm2 = _bn_relu(fm2p.reshape(BS * 256, 256), m, var, p['bn2_g'], p['bn2_b'])
    fm2 = fm2.reshape(BS, 256, 256)

    fm3p = _conv(ni2, v1, fm2, p['w3'], p['b3'].reshape(1, -1), p['dir3'],
                 N=10, Cout=256)
    m, var = _bn_last_stats(fm3p)
    fm3 = _bn_relu(fm3p.reshape(BS * 256, 256), m, var, p['bn3_g'], p['bn3_b'])
    fm3 = fm3.reshape(BS, 256, 256)

    f2 = _pool(fm3, ni2)
    v2 = v1[:, :64, :]
    v2T = v1T[:, :, :64]
    ni3 = _knn_small(v2, v2T, qc[:, :64], qr[:, :, :64], K=9)

    fm4, fglob = _conv(ni3, v2, f2, p['w4'], p['b4'].reshape(1, -1),
                       p['dir4'], N=8, Cout=512, emit_global=True)

    oh6 = jax.nn.one_hot(cat_id.reshape(-1), 6, dtype=pts.dtype)
    oh6 = oh6.reshape(BS, 1, 6)
    fm23 = jnp.concatenate([fm2, fm3], axis=2)

    x1p = _head_a(fm0, fm1, fm23, fm4, npi, oh6,
                  p['c1_w'].T, p['c1_b'].reshape(1, -1))
    m, var = _bn_mid_stats(x1p)
    x1 = _bn_relu(x1p.reshape(BS * V0, 512), m, var, p['c1_g'], p['c1_beta'])

    x2p = _mmb(x1.reshape(BS, V0, 512), p['c2_w'].T, p['c2_b'].reshape(1, -1))
    m, var = _bn_mid_stats(x2p)
    x2 = _bn_relu(x2p.reshape(BS * V0, 512), m, var, p['c2_g'], p['c2_beta'])

    x3p = _mmb(x2.reshape(BS, V0, 512), p['c3_w'].T, p['c3_b'].reshape(1, -1))
    m, var = _bn_mid_stats(x3p)
    x3 = _bn_relu(x3p.reshape(BS * V0, 256), m, var, p['c3_g'], p['c3_beta'])

    y1p = _head_c(fglob, x3.reshape(BS, V0, 256), verts,
                  p['k1_w'].T, p['k1_b'].reshape(1, -1))
    m, var = _bn_mid_stats(y1p)
    y1 = _bn_relu(y1p.reshape(BS * V0, 512), m, var, p['k1_g'], p['k1_beta'])

    y2p = _mmb(y1.reshape(BS, V0, 512), p['k2_w'].T, p['k2_b'].reshape(1, -1))
    m, var = _bn_mid_stats(y2p)
    y2 = _bn_relu(y2p.reshape(BS * V0, 256), m, var, p['k2_g'], p['k2_beta'])

    y3p = _mmb(y2.reshape(BS, V0, 256), p['k3_w'].T, p['k3_b'].reshape(1, -1))
    m, var = _bn_mid_stats(y3p)
    y3 = _bn_relu(y3p.reshape(BS * V0, 128), m, var, p['k3_g'], p['k3_beta'])

    o = _head_d(y3.reshape(BS, V0, 128), p['k4_w'].T, p['k4_b'].reshape(1, -1))
    out24 = o.reshape(BS, 40)[:, :24]
    return out24.reshape(BS, 8, 3) + mean.reshape(BS, 1, 3)
